# baseline (speedup 1.0000x reference)
import functools

import jax
import jax.numpy as jnp
from jax import lax
from jax.experimental import pallas as pl
from jax.experimental.pallas import tpu as pltpu
from jax.experimental.pallas import tpu_sc as plsc

NUM_POS = 28 * 28
EMB = 768
XD = 1536
BATCH = 64

NW = 32
B_PER_W = BATCH // NW
CHUNK = 16
NCHUNK = NUM_POS // CHUNK
LANES = 16
NVEC = EMB // LANES


def _body(x_hbm, table_hbm, out_hbm, xbuf, tbuf, sem):
    wid = lax.axis_index("s") * 2 + lax.axis_index("c")

    def chunk_body(ci, _):
        p0 = ci * CHUNK
        pltpu.async_copy(table_hbm.at[pl.ds(p0, CHUNK)], tbuf, sem).wait()

        def batch_body(k, _):
            b = wid * B_PER_W + k
            pltpu.async_copy(x_hbm.at[b, pl.ds(p0, CHUNK)], xbuf, sem).wait()

            def row_body(r, _):
                for j in range(NVEC):
                    off = j * LANES
                    v = xbuf[r, pl.ds(EMB + off, LANES)]
                    t = tbuf[r, pl.ds(off, LANES)]
                    xbuf[r, pl.ds(EMB + off, LANES)] = v + t
                return 0

            lax.fori_loop(0, CHUNK, row_body, 0)
            pltpu.async_copy(xbuf, out_hbm.at[b, pl.ds(p0, CHUNK)], sem).wait()
            return 0

        lax.fori_loop(0, B_PER_W, batch_body, 0)
        return 0

    lax.fori_loop(0, NCHUNK, chunk_body, 0)


@jax.jit
def _sc_add(x, table):
    mesh = plsc.VectorSubcoreMesh(core_axis_name="c", subcore_axis_name="s")
    f = functools.partial(
        pl.kernel,
        mesh=mesh,
        out_type=jax.ShapeDtypeStruct((BATCH, NUM_POS, XD), jnp.float32),
        scratch_types=[
            pltpu.VMEM((CHUNK, XD), jnp.float32),
            pltpu.VMEM((CHUNK, EMB), jnp.float32),
            pltpu.SemaphoreType.DMA,
        ],
    )(_body)
    return f(x, table)


def kernel(x, table):
    return _sc_add(x, table)
